# baseline (speedup 1.0000x reference)
import functools

import jax
import jax.numpy as jnp
from jax import lax
from jax.experimental import pallas as pl
from jax.experimental.pallas import tpu as pltpu
from jax.experimental.pallas import tpu_sc as plsc

_NC = 2
_NS = 16
_NW = _NC * _NS

_CH = 100
_CHP = 104


def _unpack_pair(v):
    lo = lax.bitcast_convert_type(jnp.left_shift(v, 16), jnp.float32)
    hi = lax.bitcast_convert_type(
        jnp.bitwise_and(v, jnp.int32(-65536)), jnp.float32)
    return lo, hi


def _emb_kernel(B, M, D, V):
    chunks_total = (B * M) // _CH
    chunks_per_w = chunks_total // _NW
    pairs_per_w = chunks_per_w // 2
    Dp = D // 2

    mesh = plsc.VectorSubcoreMesh(core_axis_name="c", subcore_axis_name="s")

    @functools.partial(
        pl.kernel,
        out_type=jax.ShapeDtypeStruct((chunks_total, _CH, D), jnp.float32),
        mesh=mesh,
        compiler_params=pltpu.CompilerParams(use_tc_tiling_on_sc=False),
        scratch_types=[
            pltpu.VMEM((chunks_per_w, _CHP), jnp.int32),
            pltpu.VMEM((M, Dp), jnp.int32),
            pltpu.VMEM((2, _CHP, Dp), jnp.int32),
            pltpu.VMEM((2, 2, _CH, D), jnp.float32),
            pltpu.SemaphoreType.DMA,
            pltpu.SemaphoreType.DMA,
            pltpu.SemaphoreType.DMA,
            pltpu.SemaphoreType.DMA,
        ],
    )
    def k(idx_hbm, tok_hbm, pos_hbm, out_hbm, idx_v, pos_v, ibuf_v, obuf_v,
          g0, g1, s0, s1):
        wid = lax.axis_index("s") * _NC + lax.axis_index("c")
        base = wid * chunks_per_w
        gsems = (g0, g1)
        ssems = (s0, s1)

        pltpu.sync_copy(idx_hbm.at[wid], idx_v)
        pltpu.sync_copy(pos_hbm, pos_v)

        def gather(j, b):
            return pltpu.make_async_copy(tok_hbm.at[idx_v.at[j]],
                                         ibuf_v.at[b], gsems[b])

        def store_pair(jj, pb):
            return pltpu.make_async_copy(obuf_v.at[pb],
                                         out_hbm.at[pl.ds(base + 2 * jj, 2)],
                                         ssems[pb])

        gather(0, 0).start()

        def do_pair(jj, pb):
            @pl.when(jj >= 2)
            def _():
                store_pair(jj - 2, pb).wait()

            for b in range(2):
                j = jj * 2 + b

                @pl.when(j + 1 < chunks_per_w)
                def _():
                    gather(j + 1, 1 - b).start()

                gather(j, b).wait()

                def add_rows(r2, _):
                    for dr in range(2):
                        r = r2 * 2 + dr
                        for c in range(Dp // 16):
                            s = pl.ds(c * 16, 16)
                            tlo, thi = _unpack_pair(ibuf_v[b, r, s])
                            plo, phi = _unpack_pair(pos_v[b * _CH + r, s])
                            obuf_v[pb, b, r, pl.ds(c * 16, 16)] = tlo + plo
                            obuf_v[pb, b, r, pl.ds(Dp + c * 16, 16)] = (
                                thi + phi)
                    return 0

                lax.fori_loop(0, _CH // 2, add_rows, 0)

            store_pair(jj, pb).start()

        def quad_body(q, _):
            for pb in range(2):
                do_pair(q * 2 + pb, pb)
            return 0

        lax.fori_loop(0, pairs_per_w // 2, quad_body, 0)

        store_pair(pairs_per_w - 2, 0).wait()
        store_pair(pairs_per_w - 1, 1).wait()

    return k


def _pack_halves(t):
    n, d = t.shape
    tb = t.astype(jnp.bfloat16).reshape(n, 2, 4, 16).transpose(0, 2, 3, 1)
    return lax.bitcast_convert_type(tb, jnp.int32).reshape(n, d // 2)


@jax.jit
def kernel(inputs, token_table, pos_table):
    B, M = inputs.shape
    V, D = token_table.shape
    chunks_total = (B * M) // _CH
    chunks_per_w = chunks_total // _NW

    idx = inputs.reshape(chunks_total, _CH).astype(jnp.int32)
    idx = jnp.pad(idx, ((0, 0), (0, _CHP - _CH)))
    idx = idx.reshape(_NW, chunks_per_w, _CHP)

    out = _emb_kernel(B, M, D, V)(idx, _pack_halves(token_table),
                                  _pack_halves(pos_table))
    return out.reshape(B, M, D)


# speedup vs baseline: 1.0798x; 1.0798x over previous
import functools

import jax
import jax.numpy as jnp
from jax import lax
from jax.experimental import pallas as pl
from jax.experimental.pallas import tpu as pltpu
from jax.experimental.pallas import tpu_sc as plsc

_NC = 2
_NS = 16
_NW = _NC * _NS

_CH = 100
_CHP = 104


def _unpack_pair(v):
    lo = lax.bitcast_convert_type(jnp.left_shift(v, 16), jnp.float32)
    hi = lax.bitcast_convert_type(
        jnp.bitwise_and(v, jnp.int32(-65536)), jnp.float32)
    return lo, hi


def _emb_kernel(B, M, D, V):
    chunks_total = (B * M) // _CH
    chunks_per_w = chunks_total // _NW
    pairs_per_w = chunks_per_w // 2
    Dp = D // 2

    mesh = plsc.VectorSubcoreMesh(core_axis_name="c", subcore_axis_name="s")

    @functools.partial(
        pl.kernel,
        out_type=jax.ShapeDtypeStruct((chunks_total, _CH, D), jnp.float32),
        mesh=mesh,
        compiler_params=pltpu.CompilerParams(use_tc_tiling_on_sc=False),
        scratch_types=[
            pltpu.VMEM((chunks_per_w, _CHP), jnp.int32),
            pltpu.VMEM((M, Dp), jnp.int32),
            pltpu.VMEM((2, _CHP, Dp), jnp.int32),
            pltpu.VMEM((2, 2, _CH, D), jnp.float32),
            pltpu.VMEM_SHARED((_NS, 2, _CH, D), jnp.float32),
            pltpu.SemaphoreType.DMA,
            pltpu.SemaphoreType.DMA,
            pltpu.SemaphoreType.DMA,
            pltpu.SemaphoreType.DMA,
            pltpu.SemaphoreType.DMA,
            pltpu.SemaphoreType.DMA,
        ],
    )
    def k(idx_hbm, tok_hbm, pos_hbm, out_hbm, idx_v, pos_v, ibuf_v, obuf_v,
          spm_v, g0, g1, s0, s1, c0, c1):
        wid = lax.axis_index("s") * _NC + lax.axis_index("c")
        base = wid * chunks_per_w
        gsems = (g0, g1)
        ssems = (s0, s1)
        csems = (c0, c1)
        sid = lax.axis_index("s")

        pltpu.sync_copy(idx_hbm.at[wid], idx_v)
        pltpu.sync_copy(pos_hbm, pos_v)

        def gather(j, b):
            return pltpu.make_async_copy(tok_hbm.at[idx_v.at[j]],
                                         ibuf_v.at[b], gsems[b])

        def bounce(pb, b):
            return pltpu.make_async_copy(obuf_v.at[pb, b], spm_v.at[sid, b],
                                         csems[b])

        def store_chunk(j, b):
            return pltpu.make_async_copy(spm_v.at[sid, b],
                                         out_hbm.at[base + j], ssems[b])

        gather(0, 0).start()

        def do_pair(jj, pb):
            for b in range(2):
                j = jj * 2 + b

                @pl.when(j + 1 < chunks_per_w)
                def _():
                    gather(j + 1, 1 - b).start()

                gather(j, b).wait()

                def add_rows(r2, _):
                    for dr in range(2):
                        r = r2 * 2 + dr
                        for c in range(Dp // 16):
                            s = pl.ds(c * 16, 16)
                            tlo, thi = _unpack_pair(ibuf_v[b, r, s])
                            plo, phi = _unpack_pair(pos_v[b * _CH + r, s])
                            obuf_v[pb, b, r, pl.ds(c * 16, 16)] = tlo + plo
                            obuf_v[pb, b, r, pl.ds(Dp + c * 16, 16)] = (
                                thi + phi)
                    return 0

                lax.fori_loop(0, _CH // 2, add_rows, 0)

                @pl.when(j >= 2)
                def _():
                    store_chunk(j - 2, b).wait()
                bounce(pb, b).start()
                bounce(pb, b).wait()
                store_chunk(j, b).start()

        def quad_body(q, _):
            for pb in range(2):
                do_pair(q * 2 + pb, pb)
            return 0

        lax.fori_loop(0, pairs_per_w // 2, quad_body, 0)

        store_chunk(chunks_per_w - 2, 0).wait()
        store_chunk(chunks_per_w - 1, 1).wait()

    return k


def _pack_halves(t):
    n, d = t.shape
    tb = t.astype(jnp.bfloat16).reshape(n, 2, 4, 16).transpose(0, 2, 3, 1)
    return lax.bitcast_convert_type(tb, jnp.int32).reshape(n, d // 2)


@jax.jit
def kernel(inputs, token_table, pos_table):
    B, M = inputs.shape
    V, D = token_table.shape
    chunks_total = (B * M) // _CH
    chunks_per_w = chunks_total // _NW

    idx = inputs.reshape(chunks_total, _CH).astype(jnp.int32)
    idx = jnp.pad(idx, ((0, 0), (0, _CHP - _CH)))
    idx = idx.reshape(_NW, chunks_per_w, _CHP)

    out = _emb_kernel(B, M, D, V)(idx, _pack_halves(token_table),
                                  _pack_halves(pos_table))
    return out.reshape(B, M, D)
